# baseline (speedup 1.0000x reference)
import jax
import jax.numpy as jnp
from jax import lax
from jax.experimental import pallas as pl
from jax.experimental.pallas import tpu as pltpu
from jax.experimental.pallas import tpu_sc as plsc

_B = 16384
_D = 64
_NC, _NS = 2, 16
_NW = _NC * _NS
_BPW = _B // _NW
_G = 16
_NG = _BPW // _G

_TB = 1024
_TRC = 4096


def _transpose_body(xT_ref, eye_ref, out_ref):
    out_ref[...] = lax.dot_general(
        xT_ref[...], eye_ref[...], (((0,), (0,)), ((), ())),
        preferred_element_type=jnp.float32)


def _transpose(xT, eye):
    n = xT.shape[1]
    grid = ((n + _TRC - 1) // _TRC,)
    return pl.pallas_call(
        _transpose_body,
        grid=grid,
        in_specs=[
            pl.BlockSpec((_D, _TRC), lambda i: (0, i)),
            pl.BlockSpec((_D, _D), lambda i: (0, 0)),
        ],
        out_specs=pl.BlockSpec((_TRC, _D), lambda i: (i, 0)),
        out_shape=jax.ShapeDtypeStruct((n, _D), jnp.float32),
    )(xT, eye)


def _gather_body(uid_ref, gid_ref, uf_ref, gf_ref, u_out, g_out,
                 idsu, idsg, obu, obg, sem):
    wid = lax.axis_index("s") * _NC + lax.axis_index("c")
    base = wid * _BPW
    lanes = lax.iota(jnp.int32, _G)
    pltpu.sync_copy(uid_ref.at[pl.ds(base, _BPW)], idsu)
    pltpu.sync_copy(gid_ref.at[pl.ds(base, _BPW)], idsg)

    def group(s, carry):
        off = s * _G
        iu = idsu[pl.ds(off, _G)]
        ig = idsg[pl.ds(off, _G)]
        cps = []
        for l in range(_G):
            sel = lanes == l
            idu = jnp.sum(jnp.where(sel, iu, 0))
            idg = jnp.sum(jnp.where(sel, ig, 0))
            cps.append(pltpu.async_copy(uf_ref.at[idu], obu.at[l], sem))
            cps.append(pltpu.async_copy(gf_ref.at[idg], obg.at[l], sem))
        for c in cps:
            c.wait()
        pltpu.sync_copy(obu, u_out.at[pl.ds(base + off, _G)])
        pltpu.sync_copy(obg, g_out.at[pl.ds(base + off, _G)])
        return carry

    lax.fori_loop(0, _NG, group, 0)


def _sc_gather(uids, gids, uf, gf):
    mesh = plsc.VectorSubcoreMesh(core_axis_name="c", subcore_axis_name="s")
    f = pl.kernel(
        _gather_body,
        mesh=mesh,
        compiler_params=pltpu.CompilerParams(needs_layout_passes=False),
        out_type=(jax.ShapeDtypeStruct((_B, _D), jnp.float32),
                  jax.ShapeDtypeStruct((_B, _D), jnp.float32)),
        scratch_types=[
            pltpu.VMEM((_BPW,), jnp.int32),
            pltpu.VMEM((_BPW,), jnp.int32),
            pltpu.VMEM((_G, _D), jnp.float32),
            pltpu.VMEM((_G, _D), jnp.float32),
            pltpu.SemaphoreType.DMA,
        ],
    )
    return f(uids, gids, uf, gf)


def _mlp_body(u_ref, g_ref, w1u, w1g, b1, w2, b2, w3, b3, w4, b4, w5, b5,
              w6, b6, out_ref):
    h = jnp.dot(u_ref[...], w1u[...], preferred_element_type=jnp.float32)
    h = h + jnp.dot(g_ref[...], w1g[...], preferred_element_type=jnp.float32)
    h = jnp.maximum(h + b1[...], 0.0)
    for w, b in ((w2, b2), (w3, b3), (w4, b4), (w5, b5)):
        h = jnp.maximum(
            jnp.dot(h, w[...], preferred_element_type=jnp.float32) + b[...],
            0.0)
    h = jnp.dot(h, w6[...], preferred_element_type=jnp.float32) + b6[...]
    out_ref[...] = jnp.maximum(h, 0.0)


def _mlp(u, g, w1u, w1g, b1, w2, b2, w3, b3, w4, b4, w5, b5, w6, b6):
    def full(a):
        return pl.BlockSpec(a.shape, lambda i: (0,) * a.ndim)
    grid = (_B // _TB,)
    return pl.pallas_call(
        _mlp_body,
        grid=grid,
        in_specs=[
            pl.BlockSpec((_TB, _D), lambda i: (i, 0)),
            pl.BlockSpec((_TB, _D), lambda i: (i, 0)),
            full(w1u), full(w1g), full(b1), full(w2), full(b2), full(w3),
            full(b3), full(w4), full(b4), full(w5), full(b5), full(w6),
            full(b6),
        ],
        out_specs=pl.BlockSpec((_TB, 1), lambda i: (i, 0)),
        out_shape=jax.ShapeDtypeStruct((_B, 1), jnp.float32),
    )(u, g, w1u, w1g, b1, w2, b2, w3, b3, w4, b4, w5, b5, w6, b6)


def kernel(user_ids, app_ids, user_factors, game_factors,
           W1, b1, W2, b2, W3, b3, W4, b4, W5, b5, W6, b6):
    uids = user_ids.astype(jnp.int32)
    gids = app_ids.astype(jnp.int32)
    eye = jnp.eye(_D, dtype=jnp.float32)
    uf = _transpose(user_factors.T, eye)
    gf = _transpose(game_factors.T, eye)
    u, g = _sc_gather(uids, gids, uf, gf)
    w1t = W1.T
    out = _mlp(
        u, g, w1t[:_D], w1t[_D:], b1.reshape(1, -1),
        W2.T, b2.reshape(1, -1), W3.T, b3.reshape(1, -1),
        W4.T, b4.reshape(1, -1), W5.T, b5.reshape(1, -1),
        W6.T, b6.reshape(1, -1))
    return out.reshape(_B)


# speedup vs baseline: 1.3115x; 1.3115x over previous
import jax
import jax.numpy as jnp
from jax import lax
from jax.experimental import pallas as pl
from jax.experimental.pallas import tpu as pltpu
from jax.experimental.pallas import tpu_sc as plsc

_B = 16384
_D = 64
_NC, _NS = 2, 16
_NW = _NC * _NS
_BPW = _B // _NW
_G = 16
_NG = _BPW // _G

_TB = 1024
_TRC = 16384


def _transpose_body(xT_ref, eye_ref, out_ref):
    out_ref[...] = lax.dot_general(
        xT_ref[...], eye_ref[...], (((0,), (0,)), ((), ())),
        preferred_element_type=jnp.float32)


def _transpose(xT, eye):
    n = xT.shape[1]
    grid = ((n + _TRC - 1) // _TRC,)
    return pl.pallas_call(
        _transpose_body,
        grid=grid,
        in_specs=[
            pl.BlockSpec((_D, _TRC), lambda i: (0, i)),
            pl.BlockSpec((_D, _D), lambda i: (0, 0)),
        ],
        out_specs=pl.BlockSpec((_TRC, _D), lambda i: (i, 0)),
        out_shape=jax.ShapeDtypeStruct((n, _D), jnp.float32),
    )(xT, eye)


def _gather_body(uid_ref, gid_ref, uf_ref, gf_ref, u_out, g_out,
                 idsu, idsg, obu, obg, sem):
    wid = lax.axis_index("s") * _NC + lax.axis_index("c")
    base = wid * _BPW
    lanes = lax.iota(jnp.int32, _G)
    pltpu.sync_copy(uid_ref.at[pl.ds(base, _BPW)], idsu)
    pltpu.sync_copy(gid_ref.at[pl.ds(base, _BPW)], idsg)

    def group(s, carry):
        off = s * _G
        iu = idsu[pl.ds(off, _G)]
        ig = idsg[pl.ds(off, _G)]
        cps = []
        for l in range(_G):
            sel = lanes == l
            idu = jnp.sum(jnp.where(sel, iu, 0))
            idg = jnp.sum(jnp.where(sel, ig, 0))
            cps.append(pltpu.async_copy(uf_ref.at[idu], obu.at[l], sem))
            cps.append(pltpu.async_copy(gf_ref.at[idg], obg.at[l], sem))
        for c in cps:
            c.wait()
        pltpu.sync_copy(obu, u_out.at[pl.ds(base + off, _G)])
        pltpu.sync_copy(obg, g_out.at[pl.ds(base + off, _G)])
        return carry

    lax.fori_loop(0, _NG, group, 0)


def _sc_gather(uids, gids, uf, gf):
    mesh = plsc.VectorSubcoreMesh(core_axis_name="c", subcore_axis_name="s")
    f = pl.kernel(
        _gather_body,
        mesh=mesh,
        compiler_params=pltpu.CompilerParams(needs_layout_passes=False),
        out_type=(jax.ShapeDtypeStruct((_B, _D), jnp.float32),
                  jax.ShapeDtypeStruct((_B, _D), jnp.float32)),
        scratch_types=[
            pltpu.VMEM((_BPW,), jnp.int32),
            pltpu.VMEM((_BPW,), jnp.int32),
            pltpu.VMEM((_G, _D), jnp.float32),
            pltpu.VMEM((_G, _D), jnp.float32),
            pltpu.SemaphoreType.DMA,
        ],
    )
    return f(uids, gids, uf, gf)


def _mlp_body(u_ref, g_ref, w1u, w1g, b1, w2, b2, w3, b3, w4, b4, w5, b5,
              w6, b6, out_ref):
    h = jnp.dot(u_ref[...], w1u[...], preferred_element_type=jnp.float32)
    h = h + jnp.dot(g_ref[...], w1g[...], preferred_element_type=jnp.float32)
    h = jnp.maximum(h + b1[...], 0.0)
    for w, b in ((w2, b2), (w3, b3), (w4, b4), (w5, b5)):
        h = jnp.maximum(
            jnp.dot(h, w[...], preferred_element_type=jnp.float32) + b[...],
            0.0)
    h = jnp.dot(h, w6[...], preferred_element_type=jnp.float32) + b6[...]
    out_ref[...] = jnp.maximum(h, 0.0)


def _mlp(u, g, w1u, w1g, b1, w2, b2, w3, b3, w4, b4, w5, b5, w6, b6):
    def full(a):
        return pl.BlockSpec(a.shape, lambda i: (0,) * a.ndim)
    grid = (_B // _TB,)
    return pl.pallas_call(
        _mlp_body,
        grid=grid,
        in_specs=[
            pl.BlockSpec((_TB, _D), lambda i: (i, 0)),
            pl.BlockSpec((_TB, _D), lambda i: (i, 0)),
            full(w1u), full(w1g), full(b1), full(w2), full(b2), full(w3),
            full(b3), full(w4), full(b4), full(w5), full(b5), full(w6),
            full(b6),
        ],
        out_specs=pl.BlockSpec((_TB, 1), lambda i: (i, 0)),
        out_shape=jax.ShapeDtypeStruct((_B, 1), jnp.float32),
    )(u, g, w1u, w1g, b1, w2, b2, w3, b3, w4, b4, w5, b5, w6, b6)


def kernel(user_ids, app_ids, user_factors, game_factors,
           W1, b1, W2, b2, W3, b3, W4, b4, W5, b5, W6, b6):
    uids = user_ids.astype(jnp.int32)
    gids = app_ids.astype(jnp.int32)
    eye = jnp.eye(_D, dtype=jnp.float32)
    uf = _transpose(user_factors.T, eye)
    gf = _transpose(game_factors.T, eye)
    u, g = _sc_gather(uids, gids, uf, gf)
    w1t = W1.T
    out = _mlp(
        u, g, w1t[:_D], w1t[_D:], b1.reshape(1, -1),
        W2.T, b2.reshape(1, -1), W3.T, b3.reshape(1, -1),
        W4.T, b4.reshape(1, -1), W5.T, b5.reshape(1, -1),
        W6.T, b6.reshape(1, -1))
    return out.reshape(_B)


# speedup vs baseline: 1.3441x; 1.0248x over previous
import jax
import jax.numpy as jnp
from jax import lax
from jax.experimental import pallas as pl
from jax.experimental.pallas import tpu as pltpu
from jax.experimental.pallas import tpu_sc as plsc

_B = 16384
_D = 64
_NC, _NS = 2, 16
_NW = _NC * _NS
_BPW = _B // _NW
_G = 16
_NG = _BPW // _G

_TB = 1024
_TRC = 32768


def _transpose_body(xT_ref, eye_ref, out_ref):
    out_ref[...] = lax.dot_general(
        xT_ref[...], eye_ref[...], (((0,), (0,)), ((), ())),
        preferred_element_type=jnp.float32)


def _transpose(xT, eye):
    n = xT.shape[1]
    grid = ((n + _TRC - 1) // _TRC,)
    return pl.pallas_call(
        _transpose_body,
        grid=grid,
        in_specs=[
            pl.BlockSpec((_D, _TRC), lambda i: (0, i)),
            pl.BlockSpec((_D, _D), lambda i: (0, 0)),
        ],
        out_specs=pl.BlockSpec((_TRC, _D), lambda i: (i, 0)),
        out_shape=jax.ShapeDtypeStruct((n, _D), jnp.float32),
    )(xT, eye)


def _gather_body(uid_ref, gid_ref, uf_ref, gf_ref, u_out, g_out,
                 idsu, idsg, obu, obg, sem):
    wid = lax.axis_index("s") * _NC + lax.axis_index("c")
    base = wid * _BPW
    lanes = lax.iota(jnp.int32, _G)
    pltpu.sync_copy(uid_ref.at[pl.ds(base, _BPW)], idsu)
    pltpu.sync_copy(gid_ref.at[pl.ds(base, _BPW)], idsg)

    def group(s, carry):
        off = s * _G
        iu = idsu[pl.ds(off, _G)]
        ig = idsg[pl.ds(off, _G)]
        cps = []
        for l in range(_G):
            sel = lanes == l
            idu = jnp.sum(jnp.where(sel, iu, 0))
            idg = jnp.sum(jnp.where(sel, ig, 0))
            cps.append(pltpu.async_copy(uf_ref.at[idu], obu.at[l], sem))
            cps.append(pltpu.async_copy(gf_ref.at[idg], obg.at[l], sem))
        for c in cps:
            c.wait()
        pltpu.sync_copy(obu, u_out.at[pl.ds(base + off, _G)])
        pltpu.sync_copy(obg, g_out.at[pl.ds(base + off, _G)])
        return carry

    lax.fori_loop(0, _NG, group, 0)


def _sc_gather(uids, gids, uf, gf):
    mesh = plsc.VectorSubcoreMesh(core_axis_name="c", subcore_axis_name="s")
    f = pl.kernel(
        _gather_body,
        mesh=mesh,
        compiler_params=pltpu.CompilerParams(needs_layout_passes=False),
        out_type=(jax.ShapeDtypeStruct((_B, _D), jnp.float32),
                  jax.ShapeDtypeStruct((_B, _D), jnp.float32)),
        scratch_types=[
            pltpu.VMEM((_BPW,), jnp.int32),
            pltpu.VMEM((_BPW,), jnp.int32),
            pltpu.VMEM((_G, _D), jnp.float32),
            pltpu.VMEM((_G, _D), jnp.float32),
            pltpu.SemaphoreType.DMA,
        ],
    )
    return f(uids, gids, uf, gf)


def _mlp_body(u_ref, g_ref, w1u, w1g, b1, w2, b2, w3, b3, w4, b4, w5, b5,
              w6, b6, out_ref):
    h = jnp.dot(u_ref[...], w1u[...], preferred_element_type=jnp.float32)
    h = h + jnp.dot(g_ref[...], w1g[...], preferred_element_type=jnp.float32)
    h = jnp.maximum(h + b1[...], 0.0)
    for w, b in ((w2, b2), (w3, b3), (w4, b4), (w5, b5)):
        h = jnp.maximum(
            jnp.dot(h, w[...], preferred_element_type=jnp.float32) + b[...],
            0.0)
    h = jnp.dot(h, w6[...], preferred_element_type=jnp.float32) + b6[...]
    out_ref[...] = jnp.maximum(h, 0.0)


def _mlp(u, g, w1u, w1g, b1, w2, b2, w3, b3, w4, b4, w5, b5, w6, b6):
    def full(a):
        return pl.BlockSpec(a.shape, lambda i: (0,) * a.ndim)
    grid = (_B // _TB,)
    return pl.pallas_call(
        _mlp_body,
        grid=grid,
        in_specs=[
            pl.BlockSpec((_TB, _D), lambda i: (i, 0)),
            pl.BlockSpec((_TB, _D), lambda i: (i, 0)),
            full(w1u), full(w1g), full(b1), full(w2), full(b2), full(w3),
            full(b3), full(w4), full(b4), full(w5), full(b5), full(w6),
            full(b6),
        ],
        out_specs=pl.BlockSpec((_TB, 1), lambda i: (i, 0)),
        out_shape=jax.ShapeDtypeStruct((_B, 1), jnp.float32),
    )(u, g, w1u, w1g, b1, w2, b2, w3, b3, w4, b4, w5, b5, w6, b6)


def kernel(user_ids, app_ids, user_factors, game_factors,
           W1, b1, W2, b2, W3, b3, W4, b4, W5, b5, W6, b6):
    uids = user_ids.astype(jnp.int32)
    gids = app_ids.astype(jnp.int32)
    eye = jnp.eye(_D, dtype=jnp.float32)
    uf = _transpose(user_factors.T, eye)
    gf = _transpose(game_factors.T, eye)
    u, g = _sc_gather(uids, gids, uf, gf)
    w1t = W1.T
    out = _mlp(
        u, g, w1t[:_D], w1t[_D:], b1.reshape(1, -1),
        W2.T, b2.reshape(1, -1), W3.T, b3.reshape(1, -1),
        W4.T, b4.reshape(1, -1), W5.T, b5.reshape(1, -1),
        W6.T, b6.reshape(1, -1))
    return out.reshape(_B)
